# baseline (speedup 1.0000x reference)
import jax
import jax.numpy as jnp
from jax.experimental import pallas as pl
from jax.experimental.pallas import tpu as pltpu

_INV_T = 20.0
_TILE_R = 1000


def _mm_kernel(m_ref, x_ref, o_ref):
    a = m_ref[...] * _INV_T
    o_ref[...] = jax.lax.dot_general(
        a, x_ref[...],
        dimension_numbers=(((1,), (1,)), ((), ())),
        preferred_element_type=jnp.float32)


def kernel(x, y, memory):
    del y
    b, k = x.shape
    n = memory.shape[0]
    out_t = pl.pallas_call(
        _mm_kernel,
        grid=(n // _TILE_R,),
        in_specs=[
            pl.BlockSpec((_TILE_R, k), lambda i: (i, 0)),
            pl.BlockSpec((b, k), lambda i: (0, 0)),
        ],
        out_specs=pl.BlockSpec((_TILE_R, b), lambda i: (i, 0)),
        out_shape=jax.ShapeDtypeStruct((n, b), jnp.float32),
        compiler_params=pltpu.CompilerParams(
            vmem_limit_bytes=63 * 1024 * 1024,
        ),
    )(memory, x)
    return out_t.T


# speedup vs baseline: 1.1559x; 1.1559x over previous
import jax
import jax.numpy as jnp
from jax.experimental import pallas as pl
from jax.experimental.pallas import tpu as pltpu

_INV_T = 20.0
_TILE_R = 4000


def _mm_kernel(m_ref, x_ref, o_ref):
    a = m_ref[...] * _INV_T
    o_ref[...] = jax.lax.dot_general(
        a, x_ref[...],
        dimension_numbers=(((1,), (1,)), ((), ())),
        preferred_element_type=jnp.float32)


def kernel(x, y, memory):
    del y
    b, k = x.shape
    n = memory.shape[0]
    out_t = pl.pallas_call(
        _mm_kernel,
        grid=(n // _TILE_R,),
        in_specs=[
            pl.BlockSpec((_TILE_R, k), lambda i: (i, 0)),
            pl.BlockSpec((b, k), lambda i: (0, 0)),
        ],
        out_specs=pl.BlockSpec((_TILE_R, b), lambda i: (i, 0)),
        out_shape=jax.ShapeDtypeStruct((n, b), jnp.float32),
        compiler_params=pltpu.CompilerParams(
            vmem_limit_bytes=63 * 1024 * 1024,
        ),
    )(memory, x)
    return out_t.T
